# Initial kernel scaffold; baseline (speedup 1.0000x reference)
import jax
import jax.numpy as jnp
from jax.experimental import pallas as pl


def kernel(x, x_e, edge_index, W_e, a_src_e, a_dst_e, b_e, W_h, a_src_h, a_dst_h, b_h, scale, point, tangent, w_h_w, w_e_w):
    raise NotImplementedError("write your pallas kernel here")


# speedup vs baseline: 4.4381x; 4.4381x over previous
import jax
import jax.numpy as jnp
from jax import lax
from jax.experimental import pallas as pl
from jax.experimental.pallas import tpu as pltpu
from jax.experimental.pallas import tpu_sc as plsc

N = 10000
D = 128
HW = 64
O = 40
E = 320000
NC = 2
NS = 16
NW = NC * NS
EP = E // NW
CH = 80
NCHUNK = EP // CH
NPAD = 10240
NR2 = 79
NPAD2 = NR2 * 128
ZR = 125


def _lrelu(u):
  return jnp.maximum(u, 0.2 * u)


def _prologue_body(x_ref, xe_ref, we_ref, wh_ref, ase_ref, ade_ref, ash_ref,
                   adh_ref, bh_ref, hh1_ref, hh2_ref, he_ref, scal_ref,
                   consts_ref):
  x = x_ref[...]
  xe = xe_ref[...]
  x2 = jnp.sum(x * x, axis=-1, keepdims=True)
  nrm = jnp.maximum(jnp.sqrt(x2), 1e-10)
  c = jnp.minimum(nrm, 1.0 - 1e-7)
  at = 0.5 * (jnp.log1p(c) - jnp.log1p(-c))
  lm = at * x / nrm
  hh = jnp.dot(lm, wh_ref[...], preferred_element_type=jnp.float32) + bh_ref[...]
  he = jnp.dot(xe, we_ref[...], preferred_element_type=jnp.float32)
  hh1_ref[...] = hh[:, :HW]
  hh2_ref[...] = hh[:, HW:]
  he_ref[...] = jnp.concatenate([he, jnp.zeros((N, HW - O), jnp.float32)],
                                axis=1)
  shs = jnp.dot(hh, ash_ref[...], preferred_element_type=jnp.float32)
  shd = jnp.dot(hh, adh_ref[...], preferred_element_type=jnp.float32)
  ses = jnp.dot(he, ase_ref[...], preferred_element_type=jnp.float32)
  sed = jnp.dot(he, ade_ref[...], preferred_element_type=jnp.float32)
  scal_ref[...] = jnp.concatenate(
      [shs, shd, ses, sed, x2, jnp.zeros((N, 3), jnp.float32)], axis=1)
  sh = _lrelu(jnp.max(shs) + jnp.max(shd))
  se = _lrelu(jnp.max(ses) + jnp.max(sed))
  consts_ref[...] = jnp.concatenate(
      [jnp.full((1, 16), 1.0, jnp.float32) * sh,
       jnp.full((1, 16), 1.0, jnp.float32) * se], axis=0)


def _prologue(x, x_e, W_e, W_h, a_src_e, a_dst_e, a_src_h, a_dst_h, b_h):
  f32 = jnp.float32
  return pl.pallas_call(
      _prologue_body,
      out_shape=[
          jax.ShapeDtypeStruct((N, HW), f32),
          jax.ShapeDtypeStruct((N, HW), f32),
          jax.ShapeDtypeStruct((N, HW), f32),
          jax.ShapeDtypeStruct((N, 8), f32),
          jax.ShapeDtypeStruct((2, 16), f32),
      ],
      compiler_params=pltpu.CompilerParams(vmem_limit_bytes=100 * 1024 * 1024),
  )(x, x_e, W_e, W_h,
    a_src_e.reshape(O, 1), a_dst_e.reshape(O, 1),
    a_src_h.reshape(D, 1), a_dst_h.reshape(D, 1),
    b_h.reshape(1, D))


def _newton_sqrt(a):
  a = jnp.maximum(a, 1e-30)
  i = lax.bitcast_convert_type(a, jnp.int32)
  i = 0x5F3759DF - lax.shift_right_arithmetic(i, 1)
  y = lax.bitcast_convert_type(i, jnp.float32)
  y = y * (1.5 - 0.5 * a * y * y)
  y = y * (1.5 - 0.5 * a * y * y)
  y = y * (1.5 - 0.5 * a * y * y)
  return a * y


def _sc_body(esrc, edst, shs_h, shd_h, ses_h, sed_h, x2_h, consts_h, x_h,
             hh1_h, hh2_h, he_h, oute, outh1, outh2, outdh, outde, exh_out,
             shs_v, shd_v, ses_v, sed_v, x2_v, consts_v, src_v, dst_v,
             xs_v, xd_v, row_v, exh_v, exe_v, z64, zden,
             acc_sh, denh_sh, dene_sh):
  core = lax.axis_index("c")
  sub = lax.axis_index("s")
  wid = sub * NC + core
  base = wid * EP

  pltpu.sync_copy(shs_h, shs_v)
  pltpu.sync_copy(shd_h, shd_v)
  pltpu.sync_copy(ses_h, ses_v)
  pltpu.sync_copy(sed_h, sed_v)
  pltpu.sync_copy(x2_h, x2_v)
  pltpu.sync_copy(consts_h, consts_v)

  zv = jnp.zeros((16,), jnp.float32)

  def _z64_body(i, _):
    for j in range(HW // 16):
      z64[i, pl.ds(j * 16, 16)] = zv
    return 0

  def _zden_body(i, _):
    zden[pl.ds(i * 16, 16)] = zv
    return 0

  lax.fori_loop(0, ZR, _z64_body, 0)
  lax.fori_loop(0, (NPAD // NS) // 16, _zden_body, 0)

  rstripe = N // NS

  def _zero_acc():
    def _zcopy_body(k, _):
      pltpu.sync_copy(z64, acc_sh.at[pl.ds(sub * rstripe + k * ZR, ZR)])
      return 0
    lax.fori_loop(0, rstripe // ZR, _zcopy_body, 0)

  _zero_acc()
  pltpu.sync_copy(zden, denh_sh.at[pl.ds(sub * (NPAD // NS), NPAD // NS)])
  pltpu.sync_copy(zden, dene_sh.at[pl.ds(sub * (NPAD // NS), NPAD // NS)])
  plsc.subcore_barrier()

  sh_vec = consts_v[0, :]
  se_vec = consts_v[1, :]

  def _p1_body(i, _):
    off = base + i * CH
    pltpu.sync_copy(esrc.at[pl.ds(off, CH)], src_v.at[0])
    pltpu.sync_copy(edst.at[pl.ds(off, CH)], dst_v.at[0])
    pltpu.sync_copy(x_h.at[src_v.at[0]], xs_v)
    pltpu.sync_copy(x_h.at[dst_v.at[0]], xd_v)
    pltpu.sync_copy(he_h.at[src_v.at[0]], row_v)

    for g in range(CH // 16):
      rows = jnp.arange(16, dtype=jnp.int32) + g * 16
      sv = src_v[0, pl.ds(g * 16, 16)]
      dv = dst_v[0, pl.ds(g * 16, 16)]
      svr = lax.shift_right_logical(sv, 7)
      svc = jnp.bitwise_and(sv, 127)
      dvr = lax.shift_right_logical(dv, 7)
      dvc = jnp.bitwise_and(dv, 127)
      u_h = (plsc.load_gather(shs_v, [svr, svc]) +
             plsc.load_gather(shd_v, [dvr, dvc]))
      u_e = (plsc.load_gather(ses_v, [svr, svc]) +
             plsc.load_gather(sed_v, [dvr, dvc]))
      x2s = plsc.load_gather(x2_v, [svr, svc])
      x2d = plsc.load_gather(x2_v, [dvr, dvc])

      def _dot_body(cidx, acc):
        col = jnp.full((16,), 0, jnp.int32) + cidx
        return acc + (plsc.load_gather(xs_v, [rows, col]) *
                      plsc.load_gather(xd_v, [rows, col]))

      dot = lax.fori_loop(0, D, _dot_body, jnp.zeros((16,), jnp.float32),
                          unroll=8)

      a = 1.0 - 2.0 * dot + x2d
      b = 1.0 - x2s
      den = jnp.maximum(1.0 - 2.0 * dot + x2s * x2d, 1e-10)
      n2 = jnp.maximum(a * a * x2s + b * b * x2d - 2.0 * a * b * dot, 0.0)
      r = _newton_sqrt(n2) / den
      rc = jnp.minimum(r, 1.0 - 1e-7)
      fac = (1.0 - rc) / (1.0 + rc)
      exh = jnp.exp(_lrelu(u_h) - sh_vec) * fac
      exe = jnp.exp(_lrelu(u_e) - se_vec)
      exh_v[0, pl.ds(g * 16, 16)] = exh
      exe_v[0, pl.ds(g * 16, 16)] = exe

      def _scale_body(cidx, _):
        col = jnp.full((16,), 0, jnp.int32) + cidx
        v = plsc.load_gather(row_v, [rows, col])
        plsc.store_scatter(row_v, [rows, col], v * exe)
        return 0

      lax.fori_loop(0, HW, _scale_body, 0, unroll=8)

    pltpu.sync_copy(exh_v.at[0], denh_sh.at[dst_v.at[0]], add=True)
    pltpu.sync_copy(exe_v.at[0], dene_sh.at[dst_v.at[0]], add=True)
    pltpu.sync_copy(row_v, acc_sh.at[dst_v.at[0]], add=True)
    pltpu.sync_copy(exh_v.at[0], exh_out.at[pl.ds(off, CH)])
    return 0

  lax.fori_loop(0, NCHUNK, _p1_body, 0)
  plsc.subcore_barrier()

  @pl.when(sub == 0)
  def _():
    pltpu.sync_copy(acc_sh, oute.at[core])
    pltpu.sync_copy(denh_sh, outdh.at[core])
    pltpu.sync_copy(dene_sh, outde.at[core])
  plsc.subcore_barrier()

  for hh_h, outh in ((hh1_h, outh1), (hh2_h, outh2)):
    _zero_acc()
    plsc.subcore_barrier()

    def _p_body(i, _, hh_h=hh_h):
      off = base + i * CH
      pltpu.sync_copy(esrc.at[pl.ds(off, CH)], src_v.at[0])
      pltpu.sync_copy(edst.at[pl.ds(off, CH)], dst_v.at[0])
      pltpu.sync_copy(exh_out.at[pl.ds(off, CH)], exh_v.at[0])
      pltpu.sync_copy(hh_h.at[src_v.at[0]], row_v)

      for g in range(CH // 16):
        rows = jnp.arange(16, dtype=jnp.int32) + g * 16
        exh = exh_v[0, pl.ds(g * 16, 16)]

        def _scale_body(cidx, _):
          col = jnp.full((16,), 0, jnp.int32) + cidx
          v = plsc.load_gather(row_v, [rows, col])
          plsc.store_scatter(row_v, [rows, col], v * exh)
          return 0

        lax.fori_loop(0, HW, _scale_body, 0, unroll=8)

      pltpu.sync_copy(row_v, acc_sh.at[dst_v.at[0]], add=True)
      return 0

    lax.fori_loop(0, NCHUNK, _p_body, 0)
    plsc.subcore_barrier()

    @pl.when(sub == 0)
    def _(outh=outh):
      pltpu.sync_copy(acc_sh, outh.at[core])
    plsc.subcore_barrier()


def _sc_edge_pass(esrc, edst, shs, shd, ses, sed, x2, consts, x, hh1, hh2, he):
  f32 = jnp.float32
  i32 = jnp.int32
  mesh = plsc.VectorSubcoreMesh(core_axis_name="c", subcore_axis_name="s",
                                num_cores=NC, num_subcores=NS)
  fn = pl.kernel(
      _sc_body,
      out_type=[
          jax.ShapeDtypeStruct((NC, N, HW), f32),
          jax.ShapeDtypeStruct((NC, N, HW), f32),
          jax.ShapeDtypeStruct((NC, N, HW), f32),
          jax.ShapeDtypeStruct((NC, NPAD), f32),
          jax.ShapeDtypeStruct((NC, NPAD), f32),
          jax.ShapeDtypeStruct((E,), f32),
      ],
      mesh=mesh,
      compiler_params=pltpu.CompilerParams(needs_layout_passes=False,
                                           use_tc_tiling_on_sc=False),
      scratch_types=[
          pltpu.VMEM((NR2, 128), f32),
          pltpu.VMEM((NR2, 128), f32),
          pltpu.VMEM((NR2, 128), f32),
          pltpu.VMEM((NR2, 128), f32),
          pltpu.VMEM((NR2, 128), f32),
          pltpu.VMEM((2, 16), f32),
          pltpu.VMEM((1, CH), i32),
          pltpu.VMEM((1, CH), i32),
          pltpu.VMEM((CH, D), f32),
          pltpu.VMEM((CH, D), f32),
          pltpu.VMEM((CH, HW), f32),
          pltpu.VMEM((1, CH), f32),
          pltpu.VMEM((1, CH), f32),
          pltpu.VMEM((ZR, HW), f32),
          pltpu.VMEM((NPAD // NS,), f32),
          pltpu.VMEM_SHARED((N, HW), f32),
          pltpu.VMEM_SHARED((NPAD,), f32),
          pltpu.VMEM_SHARED((NPAD,), f32),
      ],
  )
  return fn(esrc, edst, shs, shd, ses, sed, x2, consts, x, hh1, hh2, he)


def _epilogue_body(aggh_ref, rest_ref, pt_ref,
                   tg_ref, scale_ref, be_ref, whw_ref, wew_ref, out_ref):
  aggh = aggh_ref[0] + aggh_ref[1]
  combo = rest_ref[0] + rest_ref[1]
  agge = combo[:, :HW]
  denh = combo[:, HW:HW + 1]
  dene = combo[:, HW + 1:HW + 2]
  agg = jax.nn.relu(aggh / jnp.clip(denh, 1e-10))
  probs_e = agge[:, :O] / jnp.clip(dene, 1e-10) + be_ref[...]

  n2 = jnp.sum(agg * agg, axis=-1, keepdims=True)
  nrm = jnp.maximum(jnp.sqrt(n2), 1e-10)
  xh = jnp.tanh(nrm) * agg / nrm

  pt = pt_ref[...]
  tg = tg_ref[...]
  p_mat = jnp.dot(xh, pt, preferred_element_type=jnp.float32)
  t_mat = jnp.dot(xh, tg, preferred_element_type=jnp.float32)
  xn2 = jnp.sum(xh * xh, axis=-1, keepdims=True)
  pn2 = jnp.sum(pt * pt, axis=0, keepdims=True)
  ptdot = jnp.sum(pt * tg, axis=0, keepdims=True)
  an = jnp.sqrt(jnp.sum(tg * tg, axis=0, keepdims=True))
  a2 = 1.0 - 2.0 * p_mat + xn2
  b2 = 1.0 - pn2
  dd = jnp.maximum(1.0 - 2.0 * p_mat + pn2 * xn2, 1e-10)
  diff2 = (a2 * a2 * pn2 + b2 * b2 * xn2 - 2.0 * a2 * b2 * p_mat) / (dd * dd)
  inner = (b2 * t_mat - a2 * ptdot) / dd
  dend = jnp.clip(1.0 - diff2, 1e-10) * jnp.clip(an, 1e-10)
  z = 2.0 * inner / dend
  az = jnp.abs(z)
  distance = jnp.sign(z) * jnp.log(az + jnp.sqrt(az * az + 1.0))
  probs_h = distance * jnp.exp(scale_ref[...])

  xh_last = xh[N - 1:N, :]
  ln2 = jnp.sum(xh_last * xh_last, axis=-1, keepdims=True)
  lnrm = jnp.maximum(jnp.sqrt(ln2), 1e-10)
  lc = jnp.minimum(lnrm, 1.0 - 1e-7)
  lat = 0.5 * (jnp.log1p(lc) - jnp.log1p(-lc))
  lm_last = lat * xh_last / lnrm
  w_h = jax.nn.sigmoid(jnp.dot(lm_last, whw_ref[...],
                               preferred_element_type=jnp.float32))
  w_e = jax.nn.sigmoid(jnp.dot(probs_e[N - 1:N, :], wew_ref[...],
                               preferred_element_type=jnp.float32))
  s = jnp.clip(jnp.abs(w_h) + jnp.abs(w_e), 1e-10)
  out_ref[...] = (w_h / s) * probs_h + (w_e / s) * probs_e


def _epilogue(agge, aggh1, aggh2, denh, dene, point, tangent, scale, b_e,
              w_h_w, w_e_w):
  aggh_in = jnp.concatenate([aggh1, aggh2], axis=2)
  rest_in = jnp.concatenate(
      [agge, denh.reshape(NC, N, 1), dene.reshape(NC, N, 1)], axis=2)
  return pl.pallas_call(
      _epilogue_body,
      out_shape=jax.ShapeDtypeStruct((N, O), jnp.float32),
      compiler_params=pltpu.CompilerParams(vmem_limit_bytes=100 * 1024 * 1024),
  )(aggh_in, rest_in,
    point.T, tangent.T, scale.reshape(1, O), b_e.reshape(1, O),
    w_h_w, w_e_w)


def kernel(x, x_e, edge_index, W_e, a_src_e, a_dst_e, b_e, W_h, a_src_h,
           a_dst_h, b_h, scale, point, tangent, w_h_w, w_e_w):
  hh1, hh2, he, scal, consts = _prologue(
      x, x_e, W_e, W_h, a_src_e, a_dst_e, a_src_h, a_dst_h, b_h)
  esrc = edge_index[0]
  edst = edge_index[1]

  def _tab(v):
    return jnp.pad(v.reshape(N), (0, NPAD2 - N)).reshape(NR2, 128)

  agge, aggh1, aggh2, denh, dene, _ = _sc_edge_pass(
      esrc, edst, _tab(scal[:, 0]), _tab(scal[:, 1]), _tab(scal[:, 2]),
      _tab(scal[:, 3]), _tab(scal[:, 4]), consts, x, hh1, hh2, he)
  return _epilogue(agge, aggh1, aggh2, denh[:, :N], dene[:, :N], point,
                   tangent, scale, b_e, w_h_w, w_e_w)
